# baseline (speedup 1.0000x reference)
import functools

import jax
import jax.numpy as jnp
from jax import lax
from jax.experimental import pallas as pl
from jax.experimental.pallas import tpu as pltpu
from jax.experimental.pallas import tpu_sc as plsc

N = 10000
E = 320000
DH = 128
NC = 2
NS = 16
NW = NC * NS
EW = E // NW
CH = 80
NCH = 128
EWP = NCH * CH
NMSG = 4
NIDX = 8
RPW = 624
TAIL = N - NS * RPW
T = 100
S_IN = 20
S_OUT = 12
G = N // (T * S_IN)

_MESH = plsc.VectorSubcoreMesh(core_axis_name="c", subcore_axis_name="s")
_SC_PARAMS = pltpu.CompilerParams(needs_layout_passes=False)


def _hist_body(col_hbm, out_hbm, colv, histv, sem):
    c = lax.axis_index("c")
    s = lax.axis_index("s")
    w = c * NS + s
    pltpu.async_copy(col_hbm.at[w], colv, sem).wait()

    zeros16 = jnp.zeros((16,), jnp.float32)

    def zb(i, carry):
        histv[pl.ds(i * 16, 16)] = zeros16
        return carry

    lax.fori_loop(0, N // 16, zb, 0)

    ones16 = jnp.ones((16,), jnp.float32)

    def hb(i, carry):
        idx = colv[i]
        plsc.addupdate_scatter(histv, [idx], ones16)
        return carry

    lax.fori_loop(0, EW // 16, hb, 0)
    pltpu.async_copy(histv, out_hbm.at[w, 0], sem).wait()


_hist = pl.kernel(
    _hist_body,
    out_type=jax.ShapeDtypeStruct((NW, 1, N), jnp.float32),
    mesh=_MESH,
    scratch_types=[
        pltpu.VMEM((EW // 16, 16), jnp.int32),
        pltpu.VMEM((N,), jnp.float32),
        pltpu.SemaphoreType.DMA,
    ],
    compiler_params=_SC_PARAMS,
)


def _agg_body(g_hbm, gh_hbm, row_hbm, col_hbm, out_hbm,
              m0, m1, m2, m3,
              rb0, rb1, rb2, rb3, rb4, rb5, rb6, rb7,
              cb0, cb1, cb2, cb3, cb4, cb5, cb6, cb7, acc,
              gs0, gs1, gs2, gs3, ss0, ss1, ss2, ss3,
              is0, is1, is2, is3, is4, is5, is6, is7):
    c = lax.axis_index("c")
    s = lax.axis_index("s")
    w = c * NS + s
    msgs = [m0, m1, m2, m3]
    rbs = [rb0, rb1, rb2, rb3, rb4, rb5, rb6, rb7]
    cbs = [cb0, cb1, cb2, cb3, cb4, cb5, cb6, cb7]
    gss = [gs0, gs1, gs2, gs3]
    sss = [ss0, ss1, ss2, ss3]
    iss = [is0, is1, is2, is3, is4, is5, is6, is7]

    pltpu.sync_copy(gh_hbm.at[pl.ds(s * RPW, RPW)], acc.at[pl.ds(s * RPW, RPW)])

    @pl.when(s == NS - 1)
    def _():
        pltpu.sync_copy(gh_hbm.at[pl.ds(NS * RPW, TAIL)],
                        acc.at[pl.ds(NS * RPW, TAIL)])

    plsc.subcore_barrier()

    def ifire(j, q):
        pltpu.async_copy(row_hbm.at[w, j, 0], rbs[q], iss[q])
        pltpu.async_copy(col_hbm.at[w, j, 0], cbs[q], iss[q])

    def iwait(j, q):
        pltpu.make_async_copy(row_hbm.at[w, j, 0], rbs[q], iss[q]).wait()
        pltpu.make_async_copy(col_hbm.at[w, j, 0], cbs[q], iss[q]).wait()

    def gfire(b, q):
        pltpu.async_copy(g_hbm.at[rbs[q]], msgs[b], gss[b])

    def gwait(b, q):
        pltpu.make_async_copy(g_hbm.at[rbs[q]], msgs[b], gss[b]).wait()

    def sfire(b, q):
        pltpu.async_copy(msgs[b], acc.at[cbs[q]], sss[b], add=True)

    def swait(b, q):
        pltpu.make_async_copy(msgs[b], acc.at[cbs[q]], sss[b]).wait()

    for j0 in range(6):
        ifire(j0, j0)
    iwait(0, 0)
    gfire(0, 0)
    iwait(1, 1)
    gfire(1, 1)

    def step(j, bb, peel_first):
        b = bb % NMSG
        gwait(b, bb)
        sfire(b, bb)
        if not (peel_first and bb < 2):
            swait((bb + 2) % NMSG, (bb - 2) % NIDX)
        ifire(j + 6, (bb + 6) % NIDX)
        iwait(j + 2, (bb + 2) % NIDX)
        gfire((bb + 2) % NMSG, (bb + 2) % NIDX)

    for j0 in range(NIDX):
        step(j0, j0, True)

    def body(ko, carry):
        jb = ko * NIDX
        for bb in range(NIDX):
            step(jb + bb, bb, False)
        return carry

    lax.fori_loop(1, NCH // NIDX - 1, body, 0)

    for j0 in range(NCH - NIDX, NCH):
        bb = j0 % NIDX
        b = bb % NMSG
        gwait(b, bb)
        sfire(b, bb)
        swait((bb + 2) % NMSG, (bb - 2) % NIDX)
        if j0 + 6 < NCH:
            ifire(j0 + 6, (bb + 6) % NIDX)
        if j0 + 2 < NCH:
            iwait(j0 + 2, (bb + 2) % NIDX)
            gfire((bb + 2) % NMSG, (bb + 2) % NIDX)

    swait(126 % NMSG, 126 % NIDX)
    swait(127 % NMSG, 127 % NIDX)

    plsc.subcore_barrier()
    pltpu.sync_copy(acc.at[pl.ds(s * RPW, RPW)],
                    out_hbm.at[c, pl.ds(s * RPW, RPW)])

    @pl.when(s == NS - 1)
    def _():
        pltpu.sync_copy(acc.at[pl.ds(NS * RPW, TAIL)],
                        out_hbm.at[c, pl.ds(NS * RPW, TAIL)])


_agg = pl.kernel(
    _agg_body,
    out_type=jax.ShapeDtypeStruct((NC, N, DH), jnp.float32),
    mesh=_MESH,
    scratch_types=(
        [pltpu.VMEM((CH, DH), jnp.float32)] * NMSG
        + [pltpu.VMEM((CH,), jnp.int32)] * (2 * NIDX)
        + [pltpu.VMEM_SHARED((N + 8, DH), jnp.float32)]
        + [pltpu.SemaphoreType.DMA] * (2 * NMSG + NIDX)
    ),
    compiler_params=_SC_PARAMS,
)


def _dinv_body(hist_ref, dinv_ref):
    deg = jnp.sum(hist_ref[...], axis=(0, 1)) + 1.0
    dinv_ref[...] = lax.rsqrt(deg)[None, :]


def _mm0_body(x_ref, dinvT_ref, W_ref, g_ref, gh_ref):
    g = dinvT_ref[...] * jnp.dot(
        x_ref[...], W_ref[...], preferred_element_type=jnp.float32)
    g_ref[...] = g
    gh_ref[...] = 0.5 * g


def _postmm_body(p_ref, dinvT_ref, b_ref, W_ref, g_ref, gh_ref):
    dv = dinvT_ref[...]
    t = jnp.maximum(dv * (p_ref[0] + p_ref[1]) + b_ref[...], 0.0)
    g = dv * jnp.dot(t, W_ref[...], preferred_element_type=jnp.float32)
    g_ref[...] = g
    gh_ref[...] = 0.5 * g


def _pred_body(p_ref, dinvT_ref, b_ref, Wp_ref, bp_ref, out_ref):
    dv = dinvT_ref[...]
    t = jnp.maximum(dv * (p_ref[0] + p_ref[1]) + b_ref[...], 0.0)
    out_ref[...] = jnp.dot(
        t, Wp_ref[...], preferred_element_type=jnp.float32) + bp_ref[...]


_dinv = pl.pallas_call(
    _dinv_body, out_shape=jax.ShapeDtypeStruct((1, N), jnp.float32))

_mm0 = pl.pallas_call(
    _mm0_body,
    out_shape=(jax.ShapeDtypeStruct((N, DH), jnp.float32),
               jax.ShapeDtypeStruct((N, DH), jnp.float32)))

_postmm = pl.pallas_call(
    _postmm_body,
    out_shape=(jax.ShapeDtypeStruct((N, DH), jnp.float32),
               jax.ShapeDtypeStruct((N, DH), jnp.float32)))

_pred = pl.pallas_call(
    _pred_body, out_shape=jax.ShapeDtypeStruct((N, S_OUT), jnp.float32))


def kernel(x, edge_index, edge_attr, batch, W0, b0, W1, b1, W2, b2, Wp, bp):
    row3 = jnp.pad(edge_index[0].reshape(NW, EW),
                   ((0, 0), (0, EWP - EW))).reshape(NW, NCH, 1, CH)
    col3 = jnp.pad(edge_index[1].reshape(NW, EW), ((0, 0), (0, EWP - EW)),
                   constant_values=N).reshape(NW, NCH, 1, CH)
    col16 = edge_index[1].reshape(NW, EW // 16, 16)

    hist = _hist(col16)
    dinv_row = _dinv(hist)
    dinvT = dinv_row.reshape(N, 1)

    g, gh = _mm0(x, dinvT, W0)
    p = _agg(g, gh, row3, col3)
    g, gh = _postmm(p, dinvT, b0, W1)
    p = _agg(g, gh, row3, col3)
    g, gh = _postmm(p, dinvT, b1, W2)
    p = _agg(g, gh, row3, col3)
    pred = _pred(p, dinvT, b2, Wp, bp)

    out = pred.reshape(G, T * S_IN, S_OUT)[:, (S_IN - 1) * T:, :]
    return out.reshape(-1, T, S_OUT, 1)


# speedup vs baseline: 1.0003x; 1.0003x over previous
import functools

import jax
import jax.numpy as jnp
from jax import lax
from jax.experimental import pallas as pl
from jax.experimental.pallas import tpu as pltpu
from jax.experimental.pallas import tpu_sc as plsc

N = 10000
E = 320000
DH = 128
NC = 2
NS = 16
NW = NC * NS
EW = E // NW
CH = 80
NCH = 128
EWP = NCH * CH
NMSG = 4
NIDX = 8
RPW = 624
TAIL = N - NS * RPW
T = 100
S_IN = 20
S_OUT = 12
G = N // (T * S_IN)

_MESH = plsc.VectorSubcoreMesh(core_axis_name="c", subcore_axis_name="s")
_SC_PARAMS = pltpu.CompilerParams(needs_layout_passes=False)


def _hist_body(col_hbm, out_hbm, colv, histv, sem):
    c = lax.axis_index("c")
    s = lax.axis_index("s")
    w = c * NS + s
    pltpu.async_copy(col_hbm.at[w], colv, sem).wait()

    zeros16 = jnp.zeros((16,), jnp.float32)

    def zb(i, carry):
        histv[pl.ds(i * 16, 16)] = zeros16
        return carry

    lax.fori_loop(0, N // 16, zb, 0)

    ones16 = jnp.ones((16,), jnp.float32)

    def hb(i, carry):
        idx = colv[i]
        plsc.addupdate_scatter(histv, [idx], ones16)
        return carry

    lax.fori_loop(0, EW // 16, hb, 0)
    pltpu.async_copy(histv, out_hbm.at[w, 0], sem).wait()


_hist = pl.kernel(
    _hist_body,
    out_type=jax.ShapeDtypeStruct((NW, 1, N), jnp.float32),
    mesh=_MESH,
    scratch_types=[
        pltpu.VMEM((EW // 16, 16), jnp.int32),
        pltpu.VMEM((N,), jnp.float32),
        pltpu.SemaphoreType.DMA,
    ],
    compiler_params=_SC_PARAMS,
)


def _agg_body(g_hbm, gh_hbm, row_hbm, col_hbm, out_hbm,
              m0, m1, m2, m3, rg0, rg1, cg0, cg1, acc,
              gs0, gs1, gs2, gs3, ss0, ss1, ss2, ss3, is0, is1):
    c = lax.axis_index("c")
    s = lax.axis_index("s")
    w = c * NS + s
    msgs = [m0, m1, m2, m3]
    rgs = [rg0, rg1]
    cgs = [cg0, cg1]
    gss = [gs0, gs1, gs2, gs3]
    sss = [ss0, ss1, ss2, ss3]
    iss = [is0, is1]

    pltpu.sync_copy(gh_hbm.at[pl.ds(s * RPW, RPW)], acc.at[pl.ds(s * RPW, RPW)])

    @pl.when(s == NS - 1)
    def _():
        pltpu.sync_copy(gh_hbm.at[pl.ds(NS * RPW, TAIL)],
                        acc.at[pl.ds(NS * RPW, TAIL)])

    plsc.subcore_barrier()

    def ifire(gid, q):
        pltpu.async_copy(row_hbm.at[w, gid, 0], rgs[q], iss[q])
        pltpu.async_copy(col_hbm.at[w, gid], cgs[q], iss[q])

    def iwait(q):
        pltpu.make_async_copy(row_hbm.at[w, 0, 0], rgs[q], iss[q]).wait()
        pltpu.make_async_copy(col_hbm.at[w, 0], cgs[q], iss[q]).wait()

    def gfire(b, q, pos):
        pltpu.async_copy(
            g_hbm.at[rgs[q].at[pl.ds(pos * CH, CH)]], msgs[b], gss[b])

    def gwait(b, q, pos):
        pltpu.make_async_copy(
            g_hbm.at[rgs[q].at[pl.ds(pos * CH, CH)]], msgs[b], gss[b]).wait()

    def sfire(b, q, pos):
        pltpu.async_copy(msgs[b], acc.at[cgs[q].at[pos]], sss[b], add=True)

    def swait(b, q, pos):
        pltpu.make_async_copy(msgs[b], acc.at[cgs[q].at[pos]], sss[b]).wait()

    GPG = NIDX

    def group(gid, q, first, last, next_gid):
        qn = 1 - q
        for bb in range(GPG):
            b = bb % NMSG
            gwait(b, q, bb)
            sfire(b, q, bb)
            if not (first and bb < 2):
                sq = qn if bb < 2 else q
                swait((bb + 2) % NMSG, sq, (bb - 2) % GPG)
            if bb == 4 and next_gid is not None:
                ifire(next_gid, qn)
            if bb < 6:
                gfire((bb + 2) % NMSG, q, bb + 2)
            elif not last:
                if bb == 6:
                    iwait(qn)
                gfire((bb + 2) % NMSG, qn, bb - 6)

    ifire(0, 0)
    ifire(1, 1)
    iwait(0)
    gfire(0, 0, 0)
    gfire(1, 0, 1)

    group(0, 0, first=True, last=False, next_gid=None)

    def body(ko, carry):
        ga = 2 * ko + 1
        group(ga, 1, first=False, last=False, next_gid=ga + 1)
        group(ga + 1, 0, first=False, last=False, next_gid=ga + 2)
        return carry

    lax.fori_loop(0, 7, body, 0)
    group(15, 1, first=False, last=True, next_gid=None)

    swait(2, 1, 6)
    swait(3, 1, 7)

    plsc.subcore_barrier()
    pltpu.sync_copy(acc.at[pl.ds(s * RPW, RPW)],
                    out_hbm.at[c, pl.ds(s * RPW, RPW)])

    @pl.when(s == NS - 1)
    def _():
        pltpu.sync_copy(acc.at[pl.ds(NS * RPW, TAIL)],
                        out_hbm.at[c, pl.ds(NS * RPW, TAIL)])


_agg = pl.kernel(
    _agg_body,
    out_type=jax.ShapeDtypeStruct((NC, N, DH), jnp.float32),
    mesh=_MESH,
    scratch_types=(
        [pltpu.VMEM((CH, DH), jnp.float32)] * NMSG
        + [pltpu.VMEM((NIDX * CH,), jnp.int32)] * 2
        + [pltpu.VMEM((NIDX, CH), jnp.int32)] * 2
        + [pltpu.VMEM_SHARED((N + 8, DH), jnp.float32)]
        + [pltpu.SemaphoreType.DMA] * (2 * NMSG + 2)
    ),
    compiler_params=_SC_PARAMS,
)


def _dinv_body(hist_ref, dinv_ref):
    deg = jnp.sum(hist_ref[...], axis=(0, 1)) + 1.0
    dinv_ref[...] = lax.rsqrt(deg)[None, :]


def _mm0_body(x_ref, dinvT_ref, W_ref, g_ref, gh_ref):
    g = dinvT_ref[...] * jnp.dot(
        x_ref[...], W_ref[...], preferred_element_type=jnp.float32)
    g_ref[...] = g
    gh_ref[...] = 0.5 * g


def _postmm_body(p_ref, dinvT_ref, b_ref, W_ref, g_ref, gh_ref):
    dv = dinvT_ref[...]
    t = jnp.maximum(dv * (p_ref[0] + p_ref[1]) + b_ref[...], 0.0)
    g = dv * jnp.dot(t, W_ref[...], preferred_element_type=jnp.float32)
    g_ref[...] = g
    gh_ref[...] = 0.5 * g


def _pred_body(p_ref, dinvT_ref, b_ref, Wp_ref, bp_ref, out_ref):
    dv = dinvT_ref[...]
    t = jnp.maximum(dv * (p_ref[0] + p_ref[1]) + b_ref[...], 0.0)
    out_ref[...] = jnp.dot(
        t, Wp_ref[...], preferred_element_type=jnp.float32) + bp_ref[...]


_dinv = pl.pallas_call(
    _dinv_body, out_shape=jax.ShapeDtypeStruct((1, N), jnp.float32))

_mm0 = pl.pallas_call(
    _mm0_body,
    out_shape=(jax.ShapeDtypeStruct((N, DH), jnp.float32),
               jax.ShapeDtypeStruct((N, DH), jnp.float32)))

_postmm = pl.pallas_call(
    _postmm_body,
    out_shape=(jax.ShapeDtypeStruct((N, DH), jnp.float32),
               jax.ShapeDtypeStruct((N, DH), jnp.float32)))

_pred = pl.pallas_call(
    _pred_body, out_shape=jax.ShapeDtypeStruct((N, S_OUT), jnp.float32))


def kernel(x, edge_index, edge_attr, batch, W0, b0, W1, b1, W2, b2, Wp, bp):
    ngrp = NCH // NIDX
    row3 = jnp.pad(edge_index[0].reshape(NW, EW),
                   ((0, 0), (0, EWP - EW))).reshape(NW, ngrp, 1, NIDX * CH)
    col3 = jnp.pad(edge_index[1].reshape(NW, EW), ((0, 0), (0, EWP - EW)),
                   constant_values=N).reshape(NW, ngrp, NIDX, CH)
    col16 = edge_index[1].reshape(NW, EW // 16, 16)

    hist = _hist(col16)
    dinv_row = _dinv(hist)
    dinvT = dinv_row.reshape(N, 1)

    g, gh = _mm0(x, dinvT, W0)
    p = _agg(g, gh, row3, col3)
    g, gh = _postmm(p, dinvT, b0, W1)
    p = _agg(g, gh, row3, col3)
    g, gh = _postmm(p, dinvT, b1, W2)
    p = _agg(g, gh, row3, col3)
    pred = _pred(p, dinvT, b2, Wp, bp)

    out = pred.reshape(G, T * S_IN, S_OUT)[:, (S_IN - 1) * T:, :]
    return out.reshape(-1, T, S_OUT, 1)


# speedup vs baseline: 2.3204x; 2.3196x over previous
import functools

import jax
import jax.numpy as jnp
from jax import lax
from jax.experimental import pallas as pl
from jax.experimental.pallas import tpu as pltpu
from jax.experimental.pallas import tpu_sc as plsc

N = 10000
E = 320000
DH = 128
NC = 2
NS = 16
NW = NC * NS
EW = E // NW
CH = 80
NCH = EW // CH
RPW = 624
TAIL = N - NS * RPW
T = 100
S_IN = 20
S_OUT = 12
G = N // (T * S_IN)

_MESH = plsc.VectorSubcoreMesh(core_axis_name="c", subcore_axis_name="s")
_SC_PARAMS = pltpu.CompilerParams(needs_layout_passes=False)


def _hist_body(col_hbm, out_hbm, colv, histv, sem):
    c = lax.axis_index("c")
    s = lax.axis_index("s")
    w = c * NS + s
    pltpu.async_copy(col_hbm.at[w], colv, sem).wait()

    zeros16 = jnp.zeros((16,), jnp.float32)

    def zb(i, carry):
        histv[pl.ds(i * 16, 16)] = zeros16
        return carry

    lax.fori_loop(0, N // 16, zb, 0)

    ones16 = jnp.ones((16,), jnp.float32)

    def hb(i, carry):
        idx = colv[i]
        plsc.addupdate_scatter(histv, [idx], ones16)
        return carry

    lax.fori_loop(0, EW // 16, hb, 0)
    pltpu.async_copy(histv, out_hbm.at[w, 0], sem).wait()


_hist = pl.kernel(
    _hist_body,
    out_type=jax.ShapeDtypeStruct((NW, 1, N), jnp.float32),
    mesh=_MESH,
    scratch_types=[
        pltpu.VMEM((EW // 16, 16), jnp.int32),
        pltpu.VMEM((N,), jnp.float32),
        pltpu.SemaphoreType.DMA,
    ],
    compiler_params=_SC_PARAMS,
)


def _agg_body(g_hbm, gh_hbm, row_hbm, col_hbm, out_hbm, rowv, colv,
              msg0, msg1, acc, sem0, sem1, ssem0, ssem1):
    c = lax.axis_index("c")
    s = lax.axis_index("s")
    w = c * NS + s
    pltpu.async_copy(row_hbm.at[w, 0], rowv, sem0).wait()
    pltpu.async_copy(col_hbm.at[w], colv, sem0).wait()
    pltpu.sync_copy(gh_hbm.at[pl.ds(s * RPW, RPW)], acc.at[pl.ds(s * RPW, RPW)])

    @pl.when(s == NS - 1)
    def _():
        pltpu.sync_copy(gh_hbm.at[pl.ds(NS * RPW, TAIL)],
                        acc.at[pl.ds(NS * RPW, TAIL)])

    plsc.subcore_barrier()

    def gather(j, buf, sem):
        base = pl.multiple_of(j * CH, 8)
        return pltpu.async_copy(g_hbm.at[rowv.at[pl.ds(base, CH)]], buf, sem)

    def gwait(j, buf, sem):
        base = pl.multiple_of(j * CH, 8)
        pltpu.make_async_copy(g_hbm.at[rowv.at[pl.ds(base, CH)]], buf,
                              sem).wait()

    def scatter(j, buf, sem):
        return pltpu.async_copy(buf, acc.at[colv.at[j]], sem, add=True)

    def swait(j, buf, sem):
        pltpu.make_async_copy(buf, acc.at[colv.at[j]], sem).wait()

    gather(0, msg0, sem0)
    gather(1, msg1, sem1)

    def body(jo, carry):
        j = 2 * jo
        gwait(j, msg0, sem0)
        scatter(j, msg0, ssem0)
        gwait(j + 1, msg1, sem1)
        scatter(j + 1, msg1, ssem1)
        swait(j, msg0, ssem0)
        gather(j + 2, msg0, sem0)
        swait(j + 1, msg1, ssem1)

        @pl.when(j + 3 < NCH)
        def _():
            gather(j + 3, msg1, sem1)

        return carry

    lax.fori_loop(0, NCH // 2, body, 0)
    gwait(NCH - 1, msg0, sem0)
    pltpu.sync_copy(msg0, acc.at[colv.at[NCH - 1]], add=True)
    plsc.subcore_barrier()
    pltpu.sync_copy(acc.at[pl.ds(s * RPW, RPW)],
                    out_hbm.at[c, pl.ds(s * RPW, RPW)])

    @pl.when(s == NS - 1)
    def _():
        pltpu.sync_copy(acc.at[pl.ds(NS * RPW, TAIL)],
                        out_hbm.at[c, pl.ds(NS * RPW, TAIL)])


_agg = pl.kernel(
    _agg_body,
    out_type=jax.ShapeDtypeStruct((NC, N, DH), jnp.float32),
    mesh=_MESH,
    scratch_types=[
        pltpu.VMEM((EW,), jnp.int32),
        pltpu.VMEM((NCH, CH), jnp.int32),
        pltpu.VMEM((CH, DH), jnp.float32),
        pltpu.VMEM((CH, DH), jnp.float32),
        pltpu.VMEM_SHARED((N, DH), jnp.float32),
        pltpu.SemaphoreType.DMA,
        pltpu.SemaphoreType.DMA,
        pltpu.SemaphoreType.DMA,
        pltpu.SemaphoreType.DMA,
    ],
    compiler_params=_SC_PARAMS,
)


def _dinv_body(hist_ref, dinv_ref):
    deg = jnp.sum(hist_ref[...], axis=(0, 1)) + 1.0
    dinv_ref[...] = lax.rsqrt(deg)[None, :]


def _mm0_body(x_ref, dinvT_ref, W_ref, g_ref, gh_ref):
    g = dinvT_ref[...] * jnp.dot(
        x_ref[...], W_ref[...], preferred_element_type=jnp.float32)
    g_ref[...] = g
    gh_ref[...] = 0.5 * g


def _postmm_body(p_ref, dinvT_ref, b_ref, W_ref, g_ref, gh_ref):
    dv = dinvT_ref[...]
    t = jnp.maximum(dv * (p_ref[0] + p_ref[1]) + b_ref[...], 0.0)
    g = dv * jnp.dot(t, W_ref[...], preferred_element_type=jnp.float32)
    g_ref[...] = g
    gh_ref[...] = 0.5 * g


def _pred_body(p_ref, dinvT_ref, b_ref, Wp_ref, bp_ref, out_ref):
    dv = dinvT_ref[...]
    t = jnp.maximum(dv * (p_ref[0] + p_ref[1]) + b_ref[...], 0.0)
    out_ref[...] = jnp.dot(
        t, Wp_ref[...], preferred_element_type=jnp.float32) + bp_ref[...]


_dinv = pl.pallas_call(
    _dinv_body, out_shape=jax.ShapeDtypeStruct((1, N), jnp.float32))

_mm0 = pl.pallas_call(
    _mm0_body,
    out_shape=(jax.ShapeDtypeStruct((N, DH), jnp.float32),
               jax.ShapeDtypeStruct((N, DH), jnp.float32)))

_postmm = pl.pallas_call(
    _postmm_body,
    out_shape=(jax.ShapeDtypeStruct((N, DH), jnp.float32),
               jax.ShapeDtypeStruct((N, DH), jnp.float32)))

_pred = pl.pallas_call(
    _pred_body, out_shape=jax.ShapeDtypeStruct((N, S_OUT), jnp.float32))


def kernel(x, edge_index, edge_attr, batch, W0, b0, W1, b1, W2, b2, Wp, bp):
    row3 = edge_index[0].reshape(NW, 1, EW)
    col3 = edge_index[1].reshape(NW, NCH, CH)
    col16 = edge_index[1].reshape(NW, EW // 16, 16)

    hist = _hist(col16)
    dinv_row = _dinv(hist)
    dinvT = dinv_row.reshape(N, 1)

    g, gh = _mm0(x, dinvT, W0)
    p = _agg(g, gh, row3, col3)
    g, gh = _postmm(p, dinvT, b0, W1)
    p = _agg(g, gh, row3, col3)
    g, gh = _postmm(p, dinvT, b1, W2)
    p = _agg(g, gh, row3, col3)
    pred = _pred(p, dinvT, b2, Wp, bp)

    out = pred.reshape(G, T * S_IN, S_OUT)[:, (S_IN - 1) * T:, :]
    return out.reshape(-1, T, S_OUT, 1)
